# baseline (speedup 1.0000x reference)
import functools

import jax
import jax.numpy as jnp
from jax.experimental import pallas as pl
from jax.experimental.pallas import tpu as pltpu

_NUM_SKILL = 8
_TOP_K = 2
_TI = 512
_TS_SH = 1024


def _skill_kernel(idx_ref, vals_ref, x_ref, wg_ref, wu_ref, wd_ref, out_ref):
    b = pl.program_id(0)
    k = pl.program_id(2)
    i = pl.program_id(3)
    x2 = x_ref[0].astype(jnp.bfloat16)
    g = jnp.dot(x2, wg_ref[0].astype(jnp.bfloat16), preferred_element_type=jnp.float32)
    u = jnp.dot(x2, wu_ref[0].astype(jnp.bfloat16), preferred_element_type=jnp.float32)
    h = (jax.nn.gelu(g, approximate=True) * u).astype(jnp.bfloat16)
    contrib = jnp.dot(h, wd_ref[0].astype(jnp.bfloat16), preferred_element_type=jnp.float32)
    contrib = contrib * vals_ref[b, k]

    @pl.when((k == 0) & (i == 0))
    def _init():
        out_ref[0] = contrib

    @pl.when((k > 0) | (i > 0))
    def _acc():
        out_ref[0] = out_ref[0] + contrib


def _shared_kernel(x_ref, wg_ref, wu_ref, wd_ref, part_ref, out_ref):
    e = pl.program_id(2)
    i = pl.program_id(3)
    x2 = x_ref[0].astype(jnp.bfloat16)
    g = jnp.dot(x2, wg_ref[0].astype(jnp.bfloat16), preferred_element_type=jnp.float32)
    u = jnp.dot(x2, wu_ref[0].astype(jnp.bfloat16), preferred_element_type=jnp.float32)
    h = (jax.nn.gelu(g, approximate=True) * u).astype(jnp.bfloat16)
    contrib = jnp.dot(h, wd_ref[0].astype(jnp.bfloat16), preferred_element_type=jnp.float32)

    @pl.when((e == 0) & (i == 0))
    def _init():
        out_ref[0] = part_ref[0] + contrib

    @pl.when((e > 0) | (i > 0))
    def _acc():
        out_ref[0] = out_ref[0] + contrib


@jax.jit
def kernel(x, router_logits, skill_gate, skill_up, skill_down, shared_gate, shared_up, shared_down):
    B, S, H = x.shape
    E, _, I = skill_gate.shape
    E_sh = shared_gate.shape[0]
    n_i = I // _TI

    rw = jax.nn.softmax(router_logits.astype(jnp.float32), axis=-1)
    vals, idx = jax.lax.top_k(rw, _TOP_K)
    vals = (vals / jnp.sum(vals, axis=-1, keepdims=True)).astype(x.dtype)

    grid = (B, 1, _TOP_K, n_i)

    skill_out = pl.pallas_call(
        _skill_kernel,
        grid_spec=pltpu.PrefetchScalarGridSpec(
            num_scalar_prefetch=2,
            grid=grid,
            in_specs=[
                pl.BlockSpec((1, S, H), lambda b, s, k, i, idx, vals: (b, s, 0)),
                pl.BlockSpec((1, H, _TI), lambda b, s, k, i, idx, vals: (idx[b, k], 0, i)),
                pl.BlockSpec((1, H, _TI), lambda b, s, k, i, idx, vals: (idx[b, k], 0, i)),
                pl.BlockSpec((1, _TI, H), lambda b, s, k, i, idx, vals: (idx[b, k], i, 0)),
            ],
            out_specs=pl.BlockSpec((1, S, H), lambda b, s, k, i, idx, vals: (b, s, 0)),
        ),
        out_shape=jax.ShapeDtypeStruct((B, S, H), x.dtype),
    )(idx, vals, x, skill_gate, skill_up, skill_down)

    out = pl.pallas_call(
        _shared_kernel,
        grid=(B, S // _TS_SH, E_sh, n_i),
        in_specs=[
            pl.BlockSpec((1, _TS_SH, H), lambda b, s, e, i: (b, s, 0)),
            pl.BlockSpec((1, H, _TI), lambda b, s, e, i: (e, 0, i)),
            pl.BlockSpec((1, H, _TI), lambda b, s, e, i: (e, 0, i)),
            pl.BlockSpec((1, _TI, H), lambda b, s, e, i: (e, i, 0)),
            pl.BlockSpec((1, _TS_SH, H), lambda b, s, e, i: (b, s, 0)),
        ],
        out_specs=pl.BlockSpec((1, _TS_SH, H), lambda b, s, e, i: (b, s, 0)),
        out_shape=jax.ShapeDtypeStruct((B, S, H), x.dtype),
    )(x, shared_gate, shared_up, shared_down, skill_out)

    return out


# speedup vs baseline: 1.1003x; 1.1003x over previous
import jax
import jax.numpy as jnp
from jax.experimental import pallas as pl
from jax.experimental.pallas import tpu as pltpu

_TOP_K = 2
_TI = 512


def _fused_kernel(idx_ref, vals_ref, x_ref, wg_ref, wu_ref, wd_ref,
                  sg_ref, su_ref, sd_ref, out_ref):
    b = pl.program_id(0)
    k = pl.program_id(1)
    i = pl.program_id(2)

    @pl.when((k == 0) & (i == 0))
    def _init():
        out_ref[0] = jnp.zeros_like(out_ref[0])

    @pl.when(k < _TOP_K)
    def _skill():
        x2 = x_ref[0]
        g = jnp.dot(x2, wg_ref[0], preferred_element_type=jnp.float32)
        u = jnp.dot(x2, wu_ref[0], preferred_element_type=jnp.float32)
        h = jax.nn.gelu(g, approximate=True) * u * vals_ref[b, k]
        out_ref[0] += jnp.dot(h, wd_ref[0], preferred_element_type=jnp.float32)

    @pl.when(k == _TOP_K)
    def _shared():
        x2 = x_ref[0]
        g = jnp.dot(x2, sg_ref[0], preferred_element_type=jnp.float32)
        u = jnp.dot(x2, su_ref[0], preferred_element_type=jnp.float32)
        h = jax.nn.gelu(g, approximate=True) * u
        out_ref[0] += jnp.dot(h, sd_ref[0], preferred_element_type=jnp.float32)


@jax.jit
def kernel(x, router_logits, skill_gate, skill_up, skill_down, shared_gate, shared_up, shared_down):
    B, S, H = x.shape
    E, _, I = skill_gate.shape
    n_i = I // _TI

    rw = jax.nn.softmax(router_logits.astype(jnp.float32), axis=-1)
    vals, idx = jax.lax.top_k(rw, _TOP_K)
    vals = (vals / jnp.sum(vals, axis=-1, keepdims=True)).astype(x.dtype)

    def _skill_map(axis):
        def imap(b, k, i, idx, vals):
            kk = jnp.minimum(k, _TOP_K - 1)
            ii = jnp.where(k < _TOP_K, i, n_i - 1)
            e = idx[b, kk]
            return (e, 0, ii) if axis == 0 else (e, ii, 0)
        return imap

    def _shared_map(axis):
        def imap(b, k, i, idx, vals):
            ii = jnp.where(k == _TOP_K, i, 0)
            return (0, 0, ii) if axis == 0 else (0, ii, 0)
        return imap

    out = pl.pallas_call(
        _fused_kernel,
        grid_spec=pltpu.PrefetchScalarGridSpec(
            num_scalar_prefetch=2,
            grid=(B, _TOP_K + 1, n_i),
            in_specs=[
                pl.BlockSpec((1, S, H), lambda b, k, i, idx, vals: (b, 0, 0),
                             pipeline_mode=pl.Buffered(buffer_count=1)),
                pl.BlockSpec((1, H, _TI), _skill_map(0)),
                pl.BlockSpec((1, H, _TI), _skill_map(0)),
                pl.BlockSpec((1, _TI, H), _skill_map(1)),
                pl.BlockSpec((1, H, _TI), _shared_map(0)),
                pl.BlockSpec((1, H, _TI), _shared_map(0)),
                pl.BlockSpec((1, _TI, H), _shared_map(1)),
            ],
            out_specs=pl.BlockSpec((1, S, H), lambda b, k, i, idx, vals: (b, 0, 0),
                                   pipeline_mode=pl.Buffered(buffer_count=1)),
        ),
        out_shape=jax.ShapeDtypeStruct((B, S, H), x.dtype),
    )(idx, vals, x, skill_gate, skill_up, skill_down,
      shared_gate, shared_up, shared_down)

    return out
